# Initial kernel scaffold; baseline (speedup 1.0000x reference)
import jax
import jax.numpy as jnp
from jax.experimental import pallas as pl


def kernel(x, edge_index_1, edge_val_1, edge_index_2, edge_val_2, W1, b1, Wf, bf):
    raise NotImplementedError("write your pallas kernel here")


# speedup vs baseline: 3.4633x; 3.4633x over previous
import functools

import jax
import jax.numpy as jnp
from jax import lax
from jax.experimental import pallas as pl
from jax.experimental.pallas import tpu as pltpu
from jax.experimental.pallas import tpu_sc as plsc

N = 10000
H = 128
C = 40
E1 = 320000
E2 = 640000

NC = 2
NS = 16
NW = NC * NS
L = 16

B = 80
ROWS_PER_SUB = 624
TAIL_BASE = ROWS_PER_SUB * NS
TAIL_ROWS = N - TAIL_BASE
ZR = 104

_MESH = plsc.VectorSubcoreMesh(
    core_axis_name="c", subcore_axis_name="s", num_cores=NC, num_subcores=NS
)


def _zero_zbuf(zbuf):
    zero16 = jnp.zeros((L,), jnp.float32)

    def row(r, _):
        for j in range(H // L):
            zbuf[r, pl.ds(j * L, L)] = zero16
        return 0

    lax.fori_loop(0, ZR, row, 0)


def _zero_acc(zbuf, acc, sid):
    base = sid * ROWS_PER_SUB
    for k in range(ROWS_PER_SUB // ZR):
        pltpu.sync_copy(zbuf, acc.at[pl.ds(base + k * ZR, ZR)])

    @pl.when(sid == 0)
    def _tail():
        pltpu.sync_copy(zbuf.at[pl.ds(0, TAIL_ROWS)],
                        acc.at[pl.ds(TAIL_BASE, TAIL_ROWS)])


def _spmm_accumulate(x_hbm, src_hbm, dst_hbm, val_hbm, num_edges, acc,
                     srcb, dstb, valb, rows, sem, wid):
    per_tile = num_edges // NW
    nblk = per_tile // B
    base = wid * per_tile

    def blk(b, _):
        off = base + b * B
        pltpu.sync_copy(src_hbm.at[pl.ds(off, B)], srcb)
        pltpu.sync_copy(dst_hbm.at[pl.ds(off, B)], dstb)
        pltpu.sync_copy(val_hbm.at[pl.ds(off, B)], valb)
        pltpu.async_copy(x_hbm.at[srcb], rows, sem).wait()

        def scale(i, _):
            v = plsc.load_gather(valb, [jnp.full((L,), i, jnp.int32)])
            for j in range(H // L):
                rows[i, pl.ds(j * L, L)] = rows[i, pl.ds(j * L, L)] * v
            return 0

        lax.fori_loop(0, B, scale, 0)
        pltpu.sync_copy(rows, acc.at[dstb], add=True)
        return 0

    lax.fori_loop(0, nblk, blk, 0)


def _spmm_phase(x_hbm, src_hbm, dst_hbm, val_hbm, num_edges, out_hbm, acc,
                zbuf, srcb, dstb, valb, rows, sem, cid, sid, wid):
    _zero_acc(zbuf, acc, sid)
    plsc.subcore_barrier()
    _spmm_accumulate(x_hbm, src_hbm, dst_hbm, val_hbm, num_edges, acc,
                     srcb, dstb, valb, rows, sem, wid)
    plsc.subcore_barrier()
    base = sid * ROWS_PER_SUB
    pltpu.sync_copy(acc.at[pl.ds(base, ROWS_PER_SUB)],
                    out_hbm.at[cid, pl.ds(base, ROWS_PER_SUB)])

    @pl.when(sid == 0)
    def _tail():
        pltpu.sync_copy(acc.at[pl.ds(TAIL_BASE, TAIL_ROWS)],
                        out_hbm.at[cid, pl.ds(TAIL_BASE, TAIL_ROWS)])


_SC_SCRATCH = [
    pltpu.VMEM_SHARED((N, H), jnp.float32),
    pltpu.VMEM((ZR, H), jnp.float32),
    pltpu.VMEM((B,), jnp.int32),
    pltpu.VMEM((B,), jnp.int32),
    pltpu.VMEM((B,), jnp.float32),
    pltpu.VMEM((B, H), jnp.float32),
    pltpu.SemaphoreType.DMA,
]


@functools.partial(
    pl.kernel,
    out_type=(
        jax.ShapeDtypeStruct((NC, N, H), jnp.float32),
        jax.ShapeDtypeStruct((NC, N, H), jnp.float32),
    ),
    mesh=_MESH,
    scratch_types=_SC_SCRATCH,
    compiler_params=pltpu.CompilerParams(needs_layout_passes=False),
)
def _sc_layer1(x0, src1, dst1, val1, src2, dst2, val2, out1, out2,
               acc, zbuf, srcb, dstb, valb, rows, sem):
    cid = lax.axis_index("c")
    sid = lax.axis_index("s")
    wid = sid * NC + cid
    _zero_zbuf(zbuf)
    _spmm_phase(x0, src1, dst1, val1, E1, out1, acc,
                zbuf, srcb, dstb, valb, rows, sem, cid, sid, wid)
    plsc.subcore_barrier()
    _spmm_phase(x0, src2, dst2, val2, E2, out2, acc,
                zbuf, srcb, dstb, valb, rows, sem, cid, sid, wid)


@functools.partial(
    pl.kernel,
    out_type=tuple(
        jax.ShapeDtypeStruct((NC, N, H), jnp.float32) for _ in range(4)
    ),
    mesh=_MESH,
    scratch_types=_SC_SCRATCH,
    compiler_params=pltpu.CompilerParams(needs_layout_passes=False),
)
def _sc_layer2(x11, x12, src1, dst1, val1, src2, dst2, val2,
               o21, o22, o23, o24,
               acc, zbuf, srcb, dstb, valb, rows, sem):
    cid = lax.axis_index("c")
    sid = lax.axis_index("s")
    wid = sid * NC + cid
    _zero_zbuf(zbuf)
    for x_hbm, s_hbm, d_hbm, v_hbm, ne, o_hbm in (
        (x11, src1, dst1, val1, E1, o21),
        (x12, src1, dst1, val1, E1, o22),
        (x11, src2, dst2, val2, E2, o23),
        (x12, src2, dst2, val2, E2, o24),
    ):
        _spmm_phase(x_hbm, s_hbm, d_hbm, v_hbm, ne, o_hbm, acc,
                    zbuf, srcb, dstb, valb, rows, sem, cid, sid, wid)
        plsc.subcore_barrier()


ROW_BLK = 1000
GRID = N // ROW_BLK


def _mlp1_body(x_ref, w_ref, b_ref, o_ref):
    y = jnp.dot(x_ref[...], w_ref[...], preferred_element_type=jnp.float32)
    o_ref[...] = jnp.maximum(y + b_ref[...], 0.0)


def _mlp1(x, W1, b1):
    return pl.pallas_call(
        _mlp1_body,
        grid=(GRID,),
        in_specs=[
            pl.BlockSpec((ROW_BLK, H), lambda i: (i, 0)),
            pl.BlockSpec((H, H), lambda i: (0, 0)),
            pl.BlockSpec((1, H), lambda i: (0, 0)),
        ],
        out_specs=pl.BlockSpec((ROW_BLK, H), lambda i: (i, 0)),
        out_shape=jax.ShapeDtypeStruct((N, H), jnp.float32),
    )(x, W1, b1.reshape(1, H))


def _add2_body(p_ref, q_ref, o1_ref, o2_ref):
    o1_ref[...] = p_ref[0] + p_ref[1]
    o2_ref[...] = q_ref[0] + q_ref[1]


def _add_partials(p, q):
    return pl.pallas_call(
        _add2_body,
        grid=(GRID,),
        in_specs=[
            pl.BlockSpec((NC, ROW_BLK, H), lambda i: (0, i, 0)),
            pl.BlockSpec((NC, ROW_BLK, H), lambda i: (0, i, 0)),
        ],
        out_specs=[
            pl.BlockSpec((ROW_BLK, H), lambda i: (i, 0)),
            pl.BlockSpec((ROW_BLK, H), lambda i: (i, 0)),
        ],
        out_shape=[
            jax.ShapeDtypeStruct((N, H), jnp.float32),
            jax.ShapeDtypeStruct((N, H), jnp.float32),
        ],
    )(p, q)


def _final_body(x0_ref, x11_ref, x12_ref, p1_ref, p2_ref, p3_ref, p4_ref,
                wf_ref, bf_ref, o_ref):
    feats = (
        x0_ref[...],
        x11_ref[...],
        x12_ref[...],
        p1_ref[0] + p1_ref[1],
        p2_ref[0] + p2_ref[1],
        p3_ref[0] + p3_ref[1],
        p4_ref[0] + p4_ref[1],
    )
    logits = bf_ref[...]
    for k, f in enumerate(feats):
        logits = logits + jnp.dot(f, wf_ref[k],
                                  preferred_element_type=jnp.float32)
    m = jnp.max(logits, axis=1, keepdims=True)
    shifted = logits - m
    lse = jnp.log(jnp.sum(jnp.exp(shifted), axis=1, keepdims=True))
    o_ref[...] = shifted - lse


def _final(x0, x11, x12, p21, p22, p23, p24, Wf, bf):
    dense_spec = pl.BlockSpec((ROW_BLK, H), lambda i: (i, 0))
    part_spec = pl.BlockSpec((NC, ROW_BLK, H), lambda i: (0, i, 0))
    return pl.pallas_call(
        _final_body,
        grid=(GRID,),
        in_specs=[
            dense_spec, dense_spec, dense_spec,
            part_spec, part_spec, part_spec, part_spec,
            pl.BlockSpec((7, H, C), lambda i: (0, 0, 0)),
            pl.BlockSpec((1, C), lambda i: (0, 0)),
        ],
        out_specs=pl.BlockSpec((ROW_BLK, C), lambda i: (i, 0)),
        out_shape=jax.ShapeDtypeStruct((N, C), jnp.float32),
    )(x0, x11, x12, p21, p22, p23, p24, Wf.reshape(7, H, C),
      bf.reshape(1, C))


def kernel(x, edge_index_1, edge_val_1, edge_index_2, edge_val_2,
           W1, b1, Wf, bf):
    dst1, src1 = edge_index_1[0], edge_index_1[1]
    dst2, src2 = edge_index_2[0], edge_index_2[1]

    x0 = _mlp1(x, W1, b1)
    p1, p2 = _sc_layer1(x0, src1, dst1, edge_val_1, src2, dst2, edge_val_2)
    x11, x12 = _add_partials(p1, p2)
    p21, p22, p23, p24 = _sc_layer2(x11, x12, src1, dst1, edge_val_1,
                                    src2, dst2, edge_val_2)
    return _final(x0, x11, x12, p21, p22, p23, p24, Wf, bf)


# speedup vs baseline: 3.5366x; 1.0212x over previous
import functools

import jax
import jax.numpy as jnp
from jax import lax
from jax.experimental import pallas as pl
from jax.experimental.pallas import tpu as pltpu
from jax.experimental.pallas import tpu_sc as plsc

N = 10000
H = 128
C = 40
E1 = 320000
E2 = 640000

NC = 2
NS = 16
NW = NC * NS
L = 16

B = 128
NBLK1 = 80
NBLK2 = 160
ROWS_PER_SUB = 624
TAIL_BASE = ROWS_PER_SUB * NS
TAIL_ROWS = N - TAIL_BASE
ZR = 104

_MESH = plsc.VectorSubcoreMesh(
    core_axis_name="c", subcore_axis_name="s", num_cores=NC, num_subcores=NS
)


def _pad_edges(edge_index, edge_val, nblk_tile):
    e = edge_val.shape[0]
    per_tile = e // NW
    pt_pad = nblk_tile * B
    pad = ((0, 0), (0, pt_pad - per_tile))
    src = jnp.pad(edge_index[1].reshape(NW, per_tile), pad).reshape(-1)
    dst = jnp.pad(edge_index[0].reshape(NW, per_tile), pad).reshape(-1)
    val = jnp.pad(edge_val.reshape(NW, per_tile), pad).reshape(-1)
    return src, dst, val


def _zero_zbuf(zbuf):
    zero16 = jnp.zeros((L,), jnp.float32)

    def row(r, _):
        for j in range(H // L):
            zbuf[r, pl.ds(j * L, L)] = zero16
        return 0

    lax.fori_loop(0, ZR, row, 0)


def _zero_acc(zbuf, acc, sid):
    base = sid * ROWS_PER_SUB
    for k in range(ROWS_PER_SUB // ZR):
        pltpu.sync_copy(zbuf, acc.at[pl.ds(base + k * ZR, ZR)])

    @pl.when(sid == 0)
    def _tail():
        pltpu.sync_copy(zbuf.at[pl.ds(0, TAIL_ROWS)],
                        acc.at[pl.ds(TAIL_BASE, TAIL_ROWS)])


def _scale_rows(rows, valb):
    @plsc.parallel_loop(0, B, unroll=4)
    def _scale(i):
        v = plsc.load_gather(valb, [jnp.full((L,), i, jnp.int32)])
        for j in range(H // L):
            rows[i, pl.ds(j * L, L)] = rows[i, pl.ds(j * L, L)] * v


def _spmm_accumulate(x_hbm, src_hbm, dst_hbm, val_hbm, nblk, acc,
                     srcb0, dstb0, valb0, srcb1, dstb1, valb1,
                     rows0, rows1, esem0, esem1, gsem, wid):
    base = wid * nblk * B

    def issue_edges(k, srcb, dstb, valb, esem):
        off = base + k * B
        pltpu.async_copy(src_hbm.at[pl.ds(off, B)], srcb, esem)
        pltpu.async_copy(dst_hbm.at[pl.ds(off, B)], dstb, esem)
        pltpu.async_copy(val_hbm.at[pl.ds(off, B)], valb, esem)

    def wait_edges(srcb, dstb, valb, esem):
        pltpu.make_async_copy(src_hbm.at[pl.ds(0, B)], srcb, esem).wait()
        pltpu.make_async_copy(dst_hbm.at[pl.ds(0, B)], dstb, esem).wait()
        pltpu.make_async_copy(val_hbm.at[pl.ds(0, B)], valb, esem).wait()

    def issue_gather(srcb, rows):
        pltpu.async_copy(x_hbm.at[srcb], rows, gsem)

    def wait_gather(srcb, rows):
        pltpu.make_async_copy(x_hbm.at[srcb], rows, gsem).wait()

    issue_edges(0, srcb0, dstb0, valb0, esem0)
    issue_edges(1, srcb1, dstb1, valb1, esem1)
    wait_edges(srcb0, dstb0, valb0, esem0)
    issue_gather(srcb0, rows0)

    def pair(t, _):
        k0 = 2 * t
        wait_gather(srcb0, rows0)
        wait_edges(srcb1, dstb1, valb1, esem1)
        issue_gather(srcb1, rows1)
        _scale_rows(rows0, valb0)
        pltpu.sync_copy(rows0, acc.at[dstb0], add=True)

        @pl.when(k0 + 2 < nblk)
        def _more0():
            issue_edges(k0 + 2, srcb0, dstb0, valb0, esem0)

        wait_gather(srcb1, rows1)

        @pl.when(k0 + 2 < nblk)
        def _next_gather():
            wait_edges(srcb0, dstb0, valb0, esem0)
            issue_gather(srcb0, rows0)

        _scale_rows(rows1, valb1)
        pltpu.sync_copy(rows1, acc.at[dstb1], add=True)

        @pl.when(k0 + 3 < nblk)
        def _more1():
            issue_edges(k0 + 3, srcb1, dstb1, valb1, esem1)

        return 0

    lax.fori_loop(0, nblk // 2, pair, 0)


def _spmm_phase(x_hbm, src_hbm, dst_hbm, val_hbm, nblk, out_hbm, acc,
                zbuf, srcb0, dstb0, valb0, srcb1, dstb1, valb1,
                rows0, rows1, esem0, esem1, gsem, cid, sid, wid):
    _zero_acc(zbuf, acc, sid)
    plsc.subcore_barrier()
    _spmm_accumulate(x_hbm, src_hbm, dst_hbm, val_hbm, nblk, acc,
                     srcb0, dstb0, valb0, srcb1, dstb1, valb1,
                     rows0, rows1, esem0, esem1, gsem, wid)
    plsc.subcore_barrier()
    base = sid * ROWS_PER_SUB
    pltpu.sync_copy(acc.at[pl.ds(base, ROWS_PER_SUB)],
                    out_hbm.at[cid, pl.ds(base, ROWS_PER_SUB)])

    @pl.when(sid == 0)
    def _tail():
        pltpu.sync_copy(acc.at[pl.ds(TAIL_BASE, TAIL_ROWS)],
                        out_hbm.at[cid, pl.ds(TAIL_BASE, TAIL_ROWS)])


_SC_SCRATCH = [
    pltpu.VMEM_SHARED((N, H), jnp.float32),
    pltpu.VMEM((ZR, H), jnp.float32),
    pltpu.VMEM((B,), jnp.int32),
    pltpu.VMEM((B,), jnp.int32),
    pltpu.VMEM((B,), jnp.float32),
    pltpu.VMEM((B,), jnp.int32),
    pltpu.VMEM((B,), jnp.int32),
    pltpu.VMEM((B,), jnp.float32),
    pltpu.VMEM((B, H), jnp.float32),
    pltpu.VMEM((B, H), jnp.float32),
    pltpu.SemaphoreType.DMA,
    pltpu.SemaphoreType.DMA,
    pltpu.SemaphoreType.DMA,
]


@functools.partial(
    pl.kernel,
    out_type=(
        jax.ShapeDtypeStruct((NC, N, H), jnp.float32),
        jax.ShapeDtypeStruct((NC, N, H), jnp.float32),
    ),
    mesh=_MESH,
    scratch_types=_SC_SCRATCH,
    compiler_params=pltpu.CompilerParams(needs_layout_passes=False),
)
def _sc_layer1(x0, src1, dst1, val1, src2, dst2, val2, out1, out2,
               acc, zbuf, srcb0, dstb0, valb0, srcb1, dstb1, valb1,
               rows0, rows1, esem0, esem1, gsem):
    cid = lax.axis_index("c")
    sid = lax.axis_index("s")
    wid = sid * NC + cid
    _zero_zbuf(zbuf)
    bufs = (zbuf, srcb0, dstb0, valb0, srcb1, dstb1, valb1,
            rows0, rows1, esem0, esem1, gsem, cid, sid, wid)
    _spmm_phase(x0, src1, dst1, val1, NBLK1, out1, acc, *bufs)
    plsc.subcore_barrier()
    _spmm_phase(x0, src2, dst2, val2, NBLK2, out2, acc, *bufs)


@functools.partial(
    pl.kernel,
    out_type=tuple(
        jax.ShapeDtypeStruct((NC, N, H), jnp.float32) for _ in range(4)
    ),
    mesh=_MESH,
    scratch_types=_SC_SCRATCH,
    compiler_params=pltpu.CompilerParams(needs_layout_passes=False),
)
def _sc_layer2(x11, x12, src1, dst1, val1, src2, dst2, val2,
               o21, o22, o23, o24,
               acc, zbuf, srcb0, dstb0, valb0, srcb1, dstb1, valb1,
               rows0, rows1, esem0, esem1, gsem):
    cid = lax.axis_index("c")
    sid = lax.axis_index("s")
    wid = sid * NC + cid
    _zero_zbuf(zbuf)
    bufs = (zbuf, srcb0, dstb0, valb0, srcb1, dstb1, valb1,
            rows0, rows1, esem0, esem1, gsem, cid, sid, wid)
    for x_hbm, s_hbm, d_hbm, v_hbm, nblk, o_hbm in (
        (x11, src1, dst1, val1, NBLK1, o21),
        (x12, src1, dst1, val1, NBLK1, o22),
        (x11, src2, dst2, val2, NBLK2, o23),
        (x12, src2, dst2, val2, NBLK2, o24),
    ):
        _spmm_phase(x_hbm, s_hbm, d_hbm, v_hbm, nblk, o_hbm, acc, *bufs)
        plsc.subcore_barrier()


ROW_BLK = 1000
GRID = N // ROW_BLK


def _mlp1_body(x_ref, w_ref, b_ref, o_ref):
    y = jnp.dot(x_ref[...], w_ref[...], preferred_element_type=jnp.float32)
    o_ref[...] = jnp.maximum(y + b_ref[...], 0.0)


def _mlp1(x, W1, b1):
    return pl.pallas_call(
        _mlp1_body,
        grid=(GRID,),
        in_specs=[
            pl.BlockSpec((ROW_BLK, H), lambda i: (i, 0)),
            pl.BlockSpec((H, H), lambda i: (0, 0)),
            pl.BlockSpec((1, H), lambda i: (0, 0)),
        ],
        out_specs=pl.BlockSpec((ROW_BLK, H), lambda i: (i, 0)),
        out_shape=jax.ShapeDtypeStruct((N, H), jnp.float32),
    )(x, W1, b1.reshape(1, H))


def _add2_body(p_ref, q_ref, o1_ref, o2_ref):
    o1_ref[...] = p_ref[0] + p_ref[1]
    o2_ref[...] = q_ref[0] + q_ref[1]


def _add_partials(p, q):
    return pl.pallas_call(
        _add2_body,
        grid=(GRID,),
        in_specs=[
            pl.BlockSpec((NC, ROW_BLK, H), lambda i: (0, i, 0)),
            pl.BlockSpec((NC, ROW_BLK, H), lambda i: (0, i, 0)),
        ],
        out_specs=[
            pl.BlockSpec((ROW_BLK, H), lambda i: (i, 0)),
            pl.BlockSpec((ROW_BLK, H), lambda i: (i, 0)),
        ],
        out_shape=[
            jax.ShapeDtypeStruct((N, H), jnp.float32),
            jax.ShapeDtypeStruct((N, H), jnp.float32),
        ],
    )(p, q)


def _final_body(x0_ref, x11_ref, x12_ref, p1_ref, p2_ref, p3_ref, p4_ref,
                wf_ref, bf_ref, o_ref):
    feats = (
        x0_ref[...],
        x11_ref[...],
        x12_ref[...],
        p1_ref[0] + p1_ref[1],
        p2_ref[0] + p2_ref[1],
        p3_ref[0] + p3_ref[1],
        p4_ref[0] + p4_ref[1],
    )
    logits = bf_ref[...]
    for k, f in enumerate(feats):
        logits = logits + jnp.dot(f, wf_ref[k],
                                  preferred_element_type=jnp.float32)
    m = jnp.max(logits, axis=1, keepdims=True)
    shifted = logits - m
    lse = jnp.log(jnp.sum(jnp.exp(shifted), axis=1, keepdims=True))
    o_ref[...] = shifted - lse


def _final(x0, x11, x12, p21, p22, p23, p24, Wf, bf):
    dense_spec = pl.BlockSpec((ROW_BLK, H), lambda i: (i, 0))
    part_spec = pl.BlockSpec((NC, ROW_BLK, H), lambda i: (0, i, 0))
    return pl.pallas_call(
        _final_body,
        grid=(GRID,),
        in_specs=[
            dense_spec, dense_spec, dense_spec,
            part_spec, part_spec, part_spec, part_spec,
            pl.BlockSpec((7, H, C), lambda i: (0, 0, 0)),
            pl.BlockSpec((1, C), lambda i: (0, 0)),
        ],
        out_specs=pl.BlockSpec((ROW_BLK, C), lambda i: (i, 0)),
        out_shape=jax.ShapeDtypeStruct((N, C), jnp.float32),
    )(x0, x11, x12, p21, p22, p23, p24, Wf.reshape(7, H, C),
      bf.reshape(1, C))


def kernel(x, edge_index_1, edge_val_1, edge_index_2, edge_val_2,
           W1, b1, Wf, bf):
    src1, dst1, val1 = _pad_edges(edge_index_1, edge_val_1, NBLK1)
    src2, dst2, val2 = _pad_edges(edge_index_2, edge_val_2, NBLK2)

    x0 = _mlp1(x, W1, b1)
    p1, p2 = _sc_layer1(x0, src1, dst1, val1, src2, dst2, val2)
    x11, x12 = _add_partials(p1, p2)
    p21, p22, p23, p24 = _sc_layer2(x11, x12, src1, dst1, val1,
                                    src2, dst2, val2)
    return _final(x0, x11, x12, p21, p22, p23, p24, Wf, bf)
